# baseline (speedup 1.0000x reference)
import functools
import numpy as np
import jax
import jax.numpy as jnp
from jax.experimental import pallas as pl
from jax.experimental.pallas import tpu as pltpu

ALPHA = 0.2


def _ce_plain(x, j, iota0):
    up = jnp.roll(x, -j, axis=0)
    down = jnp.roll(x, j, axis=0)
    upper = (iota0 & j) != 0
    return jnp.where(upper, jnp.maximum(x, down), jnp.minimum(x, up))


def _pstride(j, nb):
    t = j.bit_length() - 1
    return 1 << (nb - 1 - t)


def _sort_columns(key, n, iota0, sgn_ref):
    nb = n.bit_length() - 1
    stored = jax.lax.bitcast_convert_type(key, jnp.float32) * sgn_ref[:, 0:1]
    k = 2
    lvl = 1
    while k <= n:
        j = k // 2
        while j >= 1:
            stored = _ce_plain(stored, _pstride(j, nb), iota0)
            j //= 2
        if k == n:
            break
        stored = stored * sgn_ref[:, lvl:lvl + 1]
        lvl += 1
        k *= 2
    return jax.lax.bitcast_convert_type(stored, jnp.int32)


def _mine_body(x_ref, w_ref, labc_ref, labr_ref, sgn_ref,
               loss_ref, cnt_ref, embs_s, sqc_s, *, blk, n, nblk):
    b = pl.program_id(0)

    @pl.when(b < nblk)
    def _():
        e = jnp.dot(x_ref[...], w_ref[...], preferred_element_type=jnp.float32)
        embs_s[pl.ds(b * blk, blk), :] = e
        sqc_s[pl.ds(b * blk, blk), :] = jnp.sum(e * e, axis=1, keepdims=True)

    @pl.when(b == 0)
    def _():
        loss_ref[...] = jnp.zeros((1, 1), jnp.float32)
        cnt_ref[...] = jnp.zeros((1, 1), jnp.float32)

    @pl.when(b >= nblk)
    def _():
        bb = b - nblk
        embs = embs_s[...]
        embs_a = embs_s[pl.ds(bb * blk, blk), :]
        g = jax.lax.dot_general(embs, embs_a, (((1,), (1,)), ((), ())),
                                preferred_element_type=jnp.float32)
        ones_row = jnp.ones((1, embs.shape[1]), jnp.float32)
        sqr = jax.lax.dot_general(ones_row, embs_a * embs_a,
                                  (((1,), (1,)), ((), ())),
                                  preferred_element_type=jnp.float32)
        d = sqc_s[...] + sqr - 2.0 * g
        d = jnp.maximum(d, 0.0)

        labc = labc_ref[...]
        labr = labr_ref[:, pl.ds(bb * blk, blk)]
        same = labc == labr
        jidx = jax.lax.broadcasted_iota(jnp.int32, (n, blk), 0)
        aidx = jax.lax.broadcasted_iota(jnp.int32, (n, blk), 1) + bb * blk
        pos = same & (aidx < jidx)
        tag = jnp.where(same, jnp.where(pos, 1, 2), 0)

        firstneg = jnp.min(jnp.where(same, n, jidx), axis=0, keepdims=True)
        firstneg = jnp.where(firstneg == n, 0, firstneg)
        fallback = jnp.sum(jnp.where(jidx == firstneg, d, 0.0),
                           axis=0, keepdims=True)

        kbits = jax.lax.bitcast_convert_type(d, jnp.int32)
        key = (kbits & ~np.int32(3)) | tag

        iota0 = jax.lax.broadcasted_iota(jnp.int32, (n, 1), 0)
        key = _sort_columns(key, n, iota0, sgn_ref)

        nb = n.bit_length() - 1
        mf = jnp.where((key & 3) == 0,
                       jax.lax.bitcast_convert_type(key, jnp.float32),
                       jnp.inf)
        bmin = mf
        for t in range(nb):
            pj = 1 << (nb - 1 - t)
            upper = (iota0 & pj) != 0
            bm = jnp.roll(bmin, -pj, axis=0)
            bp = jnp.roll(bmin, pj, axis=0)
            mf = jnp.where(upper, mf, jnp.minimum(mf, bm))
            if t < nb - 1:
                bmin = jnp.minimum(bmin, jnp.where(upper, bp, bm))

        has = mf != jnp.inf
        val = jax.lax.bitcast_convert_type(key & ~np.int32(3), jnp.float32)
        dneg = jnp.where(has, mf, fallback)
        terms = jnp.where((key & 3) == 1,
                          jnp.maximum(val - dneg + ALPHA, 0.0), 0.0)
        part = jnp.sum(terms).reshape(1, 1)
        cnt = jnp.sum(((key & 3) == 1).astype(jnp.float32)).reshape(1, 1)

        loss_ref[...] += part
        cnt_ref[...] += cnt


def kernel(batch_imgs, batch_labels, batch_titles, W):
    n, d_in = batch_imgs.shape
    d_emb = W.shape[1]
    assert (n & (n - 1)) == 0, "batch size must be a power of two"
    blk = min(128, n)
    nblk = n // blk

    labc = batch_labels.reshape(n, 1).astype(jnp.int32)
    labr = batch_labels.reshape(1, n).astype(jnp.int32)

    nb = n.bit_length() - 1
    ivec = np.array([int(format(r, f"0{nb}b")[::-1], 2) for r in range(n)])

    def _sgn(k):
        return np.where((ivec & k) != 0, -1.0, 1.0)

    cols = [_sgn(2)]
    k = 2
    while k < n:
        cols.append(_sgn(k) * _sgn(2 * k))
        k *= 2
    while len(cols) < 16:
        cols.append(np.ones(n))
    sgn = jnp.asarray(np.stack(cols, axis=1), dtype=jnp.float32)

    loss_sum, cnt = pl.pallas_call(
        functools.partial(_mine_body, blk=blk, n=n, nblk=nblk),
        grid=(2 * nblk,),
        in_specs=[
            pl.BlockSpec((blk, d_in), lambda b: (jnp.minimum(b, nblk - 1), 0)),
            pl.BlockSpec((d_in, d_emb), lambda b: (0, 0)),
            pl.BlockSpec((n, 1), lambda b: (0, 0)),
            pl.BlockSpec((1, n), lambda b: (0, 0)),
            pl.BlockSpec((n, 16), lambda b: (0, 0)),
        ],
        out_specs=[
            pl.BlockSpec((1, 1), lambda b: (0, 0)),
            pl.BlockSpec((1, 1), lambda b: (0, 0)),
        ],
        out_shape=[jax.ShapeDtypeStruct((1, 1), jnp.float32),
                   jax.ShapeDtypeStruct((1, 1), jnp.float32)],
        scratch_shapes=[
            pltpu.VMEM((n, d_emb), jnp.float32),
            pltpu.VMEM((n, 1), jnp.float32),
        ],
    )(batch_imgs, W, labc, labr, sgn)

    return loss_sum[0, 0] / jnp.maximum(cnt[0, 0], 1.0)


# speedup vs baseline: 1.1094x; 1.1094x over previous
import functools
import numpy as np
import jax
import jax.numpy as jnp
from jax.experimental import pallas as pl
from jax.experimental.pallas import tpu as pltpu

ALPHA = 0.2


def _ce_plain(x, j, iota0):
    up = jnp.roll(x, -j, axis=0)
    down = jnp.roll(x, j, axis=0)
    upper = (iota0 & j) != 0
    return jnp.where(upper, jnp.maximum(x, down), jnp.minimum(x, up))


def _pstride(j, nb):
    t = j.bit_length() - 1
    return 1 << (nb - 1 - t)


def _sort_columns(key, n, iota0, sgn_ref):
    nb = n.bit_length() - 1
    stored = jax.lax.bitcast_convert_type(key, jnp.float32) * sgn_ref[:, 0:1]
    k = 2
    lvl = 1
    while k <= n:
        j = k // 2
        while j >= 1:
            stored = _ce_plain(stored, _pstride(j, nb), iota0)
            j //= 2
        if k == n:
            break
        stored = stored * sgn_ref[:, lvl:lvl + 1]
        lvl += 1
        k *= 2
    return jax.lax.bitcast_convert_type(stored, jnp.int32)


def _mine_body(x_ref, w_ref, labc_ref, labr_ref, sgn_ref,
               loss_ref, cnt_ref, embs_s, sqc_s, *, blk, n):
    b = pl.program_id(0)

    @pl.when(b == 0)
    def _():
        e = jnp.dot(x_ref[...], w_ref[...], preferred_element_type=jnp.float32)
        embs_s[...] = e
        sqc_s[...] = jnp.sum(e * e, axis=1, keepdims=True)
        loss_ref[...] = jnp.zeros((1, 1), jnp.float32)
        cnt_ref[...] = jnp.zeros((1, 1), jnp.float32)

    embs = embs_s[...]
    embs_a = embs_s[pl.ds(b * blk, blk), :]
    g = jax.lax.dot_general(embs, embs_a, (((1,), (1,)), ((), ())),
                            preferred_element_type=jnp.float32)
    ones_row = jnp.ones((1, embs.shape[1]), jnp.float32)
    sqr = jax.lax.dot_general(ones_row, embs_a * embs_a,
                              (((1,), (1,)), ((), ())),
                              preferred_element_type=jnp.float32)
    d = sqc_s[...] + sqr - 2.0 * g
    d = jnp.maximum(d, 0.0)

    labc = labc_ref[...]
    labr = labr_ref[:, pl.ds(b * blk, blk)]
    same = labc == labr
    jidx = jax.lax.broadcasted_iota(jnp.int32, (n, blk), 0)
    aidx = jax.lax.broadcasted_iota(jnp.int32, (n, blk), 1) + b * blk
    pos = same & (aidx < jidx)
    tag = jnp.where(same, jnp.where(pos, 1, 2), 0)

    firstneg = jnp.min(jnp.where(same, n, jidx), axis=0, keepdims=True)
    firstneg = jnp.where(firstneg == n, 0, firstneg)
    fallback = jnp.sum(jnp.where(jidx == firstneg, d, 0.0),
                       axis=0, keepdims=True)

    kbits = jax.lax.bitcast_convert_type(d, jnp.int32)
    key = (kbits & ~np.int32(3)) | tag

    iota0 = jax.lax.broadcasted_iota(jnp.int32, (n, 1), 0)
    key = _sort_columns(key, n, iota0, sgn_ref)

    nb = n.bit_length() - 1
    mf = jnp.where((key & 3) == 0,
                   jax.lax.bitcast_convert_type(key, jnp.float32), jnp.inf)
    bmin = mf
    for t in range(nb):
        pj = 1 << (nb - 1 - t)
        upper = (iota0 & pj) != 0
        bm = jnp.roll(bmin, -pj, axis=0)
        bp = jnp.roll(bmin, pj, axis=0)
        mf = jnp.where(upper, mf, jnp.minimum(mf, bm))
        if t < nb - 1:
            bmin = jnp.minimum(bmin, jnp.where(upper, bp, bm))

    has = mf != jnp.inf
    val = jax.lax.bitcast_convert_type(key & ~np.int32(3), jnp.float32)
    dneg = jnp.where(has, mf, fallback)
    terms = jnp.where((key & 3) == 1,
                      jnp.maximum(val - dneg + ALPHA, 0.0), 0.0)
    part = jnp.sum(terms).reshape(1, 1)
    cnt = jnp.sum(((key & 3) == 1).astype(jnp.float32)).reshape(1, 1)

    loss_ref[...] += part
    cnt_ref[...] += cnt


def kernel(batch_imgs, batch_labels, batch_titles, W):
    n, d_in = batch_imgs.shape
    d_emb = W.shape[1]
    assert (n & (n - 1)) == 0, "batch size must be a power of two"
    blk = min(256, n)
    nblk = n // blk

    labc = batch_labels.reshape(n, 1).astype(jnp.int32)
    labr = batch_labels.reshape(1, n).astype(jnp.int32)

    nb = n.bit_length() - 1
    ivec = np.array([int(format(r, f"0{nb}b")[::-1], 2) for r in range(n)])

    def _sgn(k):
        return np.where((ivec & k) != 0, -1.0, 1.0)

    cols = [_sgn(2)]
    k = 2
    while k < n:
        cols.append(_sgn(k) * _sgn(2 * k))
        k *= 2
    while len(cols) < 16:
        cols.append(np.ones(n))
    sgn = jnp.asarray(np.stack(cols, axis=1), dtype=jnp.float32)

    loss_sum, cnt = pl.pallas_call(
        functools.partial(_mine_body, blk=blk, n=n),
        grid=(nblk,),
        in_specs=[
            pl.BlockSpec((n, d_in), lambda b: (0, 0)),
            pl.BlockSpec((d_in, d_emb), lambda b: (0, 0)),
            pl.BlockSpec((n, 1), lambda b: (0, 0)),
            pl.BlockSpec((1, n), lambda b: (0, 0)),
            pl.BlockSpec((n, 16), lambda b: (0, 0)),
        ],
        out_specs=[
            pl.BlockSpec((1, 1), lambda b: (0, 0)),
            pl.BlockSpec((1, 1), lambda b: (0, 0)),
        ],
        out_shape=[jax.ShapeDtypeStruct((1, 1), jnp.float32),
                   jax.ShapeDtypeStruct((1, 1), jnp.float32)],
        scratch_shapes=[
            pltpu.VMEM((n, d_emb), jnp.float32),
            pltpu.VMEM((n, 1), jnp.float32),
        ],
    )(batch_imgs, W, labc, labr, sgn)

    return loss_sum[0, 0] / jnp.maximum(cnt[0, 0], 1.0)
